# baseline (speedup 1.0000x reference)
import jax
import jax.numpy as jnp
from jax.experimental import pallas as pl
from jax.experimental.pallas import tpu as pltpu
from jax.scipy.special import gammaln

_B, _L, _D = 4, 2048, 128
_TEMP = 1.0
_THRESHOLD = 0.5
_PRIOR = 0.2
_EPS = 1e-6
_SBLK = 512


def _seg_kernel(hid_ref, wq_ref, wk_ref, u_ref, c_ref, nb_ref):
    h = hid_ref[0]
    norm = jnp.sqrt(jnp.sum(h * h, axis=1, keepdims=True))
    nrm = h / jnp.maximum(norm, 1e-12)
    m = jax.lax.dot_general(wq_ref[...], wk_ref[...], (((0,), (0,)), ((), ())),
                            preferred_element_type=jnp.float32)
    a = jnp.dot(nrm, m, preferred_element_type=jnp.float32)
    cs = jnp.sum(a[:-1] * nrm[1:], axis=1, keepdims=True)
    cos = jnp.concatenate(
        [jnp.full((1, 1), -1.0, jnp.float32), cs], axis=0)
    probs = jnp.clip((1.0 - cos) * 0.5, 0.0, 1.0)
    p = jnp.clip(probs, _EPS, 1.0 - _EPS)
    logits = jnp.log(p) - jnp.log1p(-p)
    u = u_ref[0]
    noise = jnp.log(u) - jnp.log1p(-u)
    soft = jax.nn.sigmoid((logits + noise) / _TEMP)
    hard = (soft > _THRESHOLD).astype(jnp.float32)
    row = jax.lax.broadcasted_iota(jnp.int32, (_L, _L), 0)
    col = jax.lax.broadcasted_iota(jnp.int32, (_L, _L), 1)
    tri = (col <= row).astype(jnp.float32)
    cinc = jax.lax.dot_general(tri, hard, (((1,), (0,)), ((), ())),
                               preferred_element_type=jnp.float32)
    c_ref[0] = cinc - hard
    nb_ref[0] = jnp.sum(hard, axis=0, keepdims=True)


def _pool_kernel(hid_ref, c_ref, out_ref):
    h = hid_ref[0]
    cl = c_ref[0]
    sb = pl.program_id(1)
    srange = (sb * _SBLK
              + jax.lax.broadcasted_iota(jnp.int32, (1, _SBLK), 1)
              ).astype(jnp.float32)
    mask = (cl == srange).astype(jnp.float32)
    acc = jax.lax.dot_general(mask, h, (((0,), (0,)), ((), ())),
                              preferred_element_type=jnp.float32)
    ones = jnp.ones((_L, 1), jnp.float32)
    cnt = jax.lax.dot_general(mask, ones, (((0,), (0,)), ((), ())),
                              preferred_element_type=jnp.float32)
    out_ref[0] = acc / (cnt + 1e-9)


def _binom_loss(num_b, total_p, prior):
    logp = (gammaln(total_p + 1.0) - gammaln(num_b + 1.0)
            - gammaln(total_p - num_b + 1.0)
            + num_b * jnp.log(prior) + (total_p - num_b) * jnp.log1p(-prior))
    return -logp / total_p


def kernel(hidden, Wq, Wk, u):
    b, l, d = hidden.shape
    u3 = u.reshape(b, l, 1)
    c, nb = pl.pallas_call(
        _seg_kernel,
        grid=(b,),
        in_specs=[
            pl.BlockSpec((1, l, d), lambda i: (i, 0, 0)),
            pl.BlockSpec((d, d), lambda i: (0, 0)),
            pl.BlockSpec((d, d), lambda i: (0, 0)),
            pl.BlockSpec((1, l, 1), lambda i: (i, 0, 0)),
        ],
        out_specs=[
            pl.BlockSpec((1, l, 1), lambda i: (i, 0, 0)),
            pl.BlockSpec((1, 1, 1), lambda i: (i, 0, 0)),
        ],
        out_shape=[
            jax.ShapeDtypeStruct((b, l, 1), jnp.float32),
            jax.ShapeDtypeStruct((b, 1, 1), jnp.float32),
        ],
        compiler_params=pltpu.CompilerParams(
            dimension_semantics=("parallel",)),
    )(hidden, Wq, Wk, u3)

    pooled = pl.pallas_call(
        _pool_kernel,
        grid=(b, l // _SBLK),
        in_specs=[
            pl.BlockSpec((1, l, d), lambda i, j: (i, 0, 0)),
            pl.BlockSpec((1, l, 1), lambda i, j: (i, 0, 0)),
        ],
        out_specs=pl.BlockSpec((1, _SBLK, d), lambda i, j: (i, j, 0)),
        out_shape=jax.ShapeDtypeStruct((b, l, d), jnp.float32),
        compiler_params=pltpu.CompilerParams(
            dimension_semantics=("parallel", "parallel")),
    )(hidden, c)

    num_boundaries = jnp.sum(nb)
    total_positions = jnp.float32(b * l)
    loss = _binom_loss(num_boundaries, total_positions, _PRIOR)
    return pooled, loss, num_boundaries, total_positions


# speedup vs baseline: 1.1775x; 1.1775x over previous
import jax
import jax.numpy as jnp
from jax.experimental import pallas as pl
from jax.experimental.pallas import tpu as pltpu
from jax.scipy.special import gammaln

_B, _L, _D = 4, 2048, 128
_PRIOR = 0.2
_EPS = 1e-6
_SBLK = 512
_CHUNK = 256


def _fused_kernel(hid_ref, wq_ref, wk_ref, u_ref, out_ref, nb_ref,
                  c_scr, acc_scr, cnt_scr):
    h = hid_ref[0]
    norm = jnp.sqrt(jnp.sum(h * h, axis=1, keepdims=True))
    nrm = h / jnp.maximum(norm, 1e-12)
    m = jax.lax.dot_general(wq_ref[...], wk_ref[...], (((0,), (0,)), ((), ())),
                            preferred_element_type=jnp.float32)
    a = jnp.dot(nrm, m, preferred_element_type=jnp.float32)
    cs = jnp.sum(a[:-1] * nrm[1:], axis=1, keepdims=True)
    cos = jnp.concatenate(
        [jnp.full((1, 1), -1.0, jnp.float32), cs], axis=0)
    probs = jnp.clip((1.0 - cos) * 0.5, 0.0, 1.0)
    p = jnp.clip(probs, _EPS, 1.0 - _EPS)
    u = u_ref[0]
    hard = ((p + u) > 1.0).astype(jnp.float32)
    row = jax.lax.broadcasted_iota(jnp.int32, (_L, _L), 0)
    col = jax.lax.broadcasted_iota(jnp.int32, (_L, _L), 1)
    tri = (col <= row).astype(jnp.float32)
    cinc = jax.lax.dot_general(tri, hard, (((1,), (0,)), ((), ())),
                               preferred_element_type=jnp.float32)
    c_scr[...] = cinc - hard
    nb_ref[0] = jnp.sum(hard, axis=0, keepdims=True)

    ones = jnp.ones((_CHUNK, 1), jnp.float32)
    lane = jax.lax.broadcasted_iota(jnp.int32, (1, _SBLK), 1)

    def tile(ch, s_lo):
        c_last = c_scr[(ch + 1) * _CHUNK - 1, 0]
        c_first = c_scr[ch * _CHUNK, 0]

        @pl.when((c_last >= s_lo) & (c_first < s_lo + _SBLK))
        def _():
            cl = c_scr[ch * _CHUNK:(ch + 1) * _CHUNK, :]
            srange = (s_lo + lane).astype(jnp.float32)
            mask = (cl == srange).astype(jnp.float32)
            hc = h[ch * _CHUNK:(ch + 1) * _CHUNK, :]
            acc_scr[...] += jax.lax.dot_general(
                mask, hc, (((0,), (0,)), ((), ())),
                preferred_element_type=jnp.float32)
            cnt_scr[...] += jax.lax.dot_general(
                mask, ones, (((0,), (0,)), ((), ())),
                preferred_element_type=jnp.float32)

    for sb in range(_L // _SBLK):
        s_lo = sb * _SBLK
        acc_scr[...] = jnp.zeros((_SBLK, _D), jnp.float32)
        cnt_scr[...] = jnp.zeros((_SBLK, 1), jnp.float32)
        for ch in range(_L // _CHUNK):
            tile(ch, s_lo)
        out_ref[0, s_lo:s_lo + _SBLK, :] = (
            acc_scr[...] / (cnt_scr[...] + 1e-9))


def _binom_loss(num_b, total_p, prior):
    logp = (gammaln(total_p + 1.0) - gammaln(num_b + 1.0)
            - gammaln(total_p - num_b + 1.0)
            + num_b * jnp.log(prior) + (total_p - num_b) * jnp.log1p(-prior))
    return -logp / total_p


def kernel(hidden, Wq, Wk, u):
    b, l, d = hidden.shape
    u3 = u.reshape(b, l, 1)
    pooled, nb = pl.pallas_call(
        _fused_kernel,
        grid=(b,),
        in_specs=[
            pl.BlockSpec((1, l, d), lambda i: (i, 0, 0)),
            pl.BlockSpec((d, d), lambda i: (0, 0)),
            pl.BlockSpec((d, d), lambda i: (0, 0)),
            pl.BlockSpec((1, l, 1), lambda i: (i, 0, 0)),
        ],
        out_specs=[
            pl.BlockSpec((1, l, d), lambda i: (i, 0, 0)),
            pl.BlockSpec((1, 1, 1), lambda i: (i, 0, 0)),
        ],
        out_shape=[
            jax.ShapeDtypeStruct((b, l, d), jnp.float32),
            jax.ShapeDtypeStruct((b, 1, 1), jnp.float32),
        ],
        scratch_shapes=[
            pltpu.VMEM((l, 1), jnp.float32),
            pltpu.VMEM((_SBLK, d), jnp.float32),
            pltpu.VMEM((_SBLK, 1), jnp.float32),
        ],
        compiler_params=pltpu.CompilerParams(
            dimension_semantics=("parallel",)),
    )(hidden, Wq, Wk, u3)

    num_boundaries = jnp.sum(nb)
    total_positions = jnp.float32(b * l)
    loss = _binom_loss(num_boundaries, total_positions, _PRIOR)
    return pooled, loss, num_boundaries, total_positions


# speedup vs baseline: 2.0867x; 1.7721x over previous
import jax
import jax.numpy as jnp
from jax.experimental import pallas as pl
from jax.experimental.pallas import tpu as pltpu
from jax.scipy.special import gammaln

_B, _L, _D = 4, 2048, 128
_PRIOR = 0.2
_EPS = 1e-6
_CHUNK = 256
_WIN = 384


def _fused_kernel(hid_ref, wq_ref, wk_ref, u_ref, out_ref, nb_ref,
                  c_scr, cnt_scr):
    h = hid_ref[0]
    ones_d = jnp.ones((_D, 1), jnp.float32)
    sumsq = jax.lax.dot_general(h * h, ones_d, (((1,), (0,)), ((), ())),
                                preferred_element_type=jnp.float32)
    nrm = h * (1.0 / jnp.maximum(jnp.sqrt(sumsq), 1e-12))
    m = jax.lax.dot_general(wq_ref[...], wk_ref[...], (((0,), (0,)), ((), ())),
                            preferred_element_type=jnp.float32)
    a = jnp.dot(nrm, m, preferred_element_type=jnp.float32)
    cs = jax.lax.dot_general(a[:-1] * nrm[1:], ones_d, (((1,), (0,)), ((), ())),
                             preferred_element_type=jnp.float32)
    cos = jnp.concatenate(
        [jnp.full((1, 1), -1.0, jnp.float32), cs], axis=0)
    probs = jnp.clip((1.0 - cos) * 0.5, 0.0, 1.0)
    p = jnp.clip(probs, _EPS, 1.0 - _EPS)
    u = u_ref[0]
    hard = ((p + u) > 1.0).astype(jnp.float32)
    nck = _L // 128
    row_b = jax.lax.broadcasted_iota(jnp.int32, (nck, _L), 0)
    col_b = jax.lax.broadcasted_iota(jnp.int32, (nck, _L), 1)
    bsel = ((col_b >> 7) == row_b).astype(jnp.float32)
    tot = jax.lax.dot_general(bsel, hard, (((1,), (0,)), ((), ())),
                              preferred_element_type=jnp.float32)
    row_e = jax.lax.broadcasted_iota(jnp.int32, (_L, nck), 0)
    col_e = jax.lax.broadcasted_iota(jnp.int32, (_L, nck), 1)
    esel = (col_e < (row_e >> 7)).astype(jnp.float32)
    coarse = jax.lax.dot_general(esel, tot, (((1,), (0,)), ((), ())),
                                 preferred_element_type=jnp.float32)
    r1 = jax.lax.broadcasted_iota(jnp.int32, (128, 128), 0)
    c1 = jax.lax.broadcasted_iota(jnp.int32, (128, 128), 1)
    tri = (c1 <= r1).astype(jnp.float32)
    within = jnp.concatenate(
        [jax.lax.dot_general(tri, hard[128 * i:128 * (i + 1), :],
                             (((1,), (0,)), ((), ())),
                             preferred_element_type=jnp.float32)
         for i in range(nck)], axis=0)
    c_scr[...] = within + coarse - hard
    nb_ref[0] = jnp.sum(hard, axis=0, keepdims=True)

    out_ref[0] = jnp.zeros((_L, _D), jnp.float32)
    cnt_scr[...] = jnp.zeros((_L, 1), jnp.float32)
    ones_c = jnp.ones((_CHUNK, 1), jnp.bfloat16)
    lane = jax.lax.broadcasted_iota(jnp.int32, (1, _WIN), 1)

    for ch in range(_L // _CHUNK):
        s_base = c_scr[ch * _CHUNK, 0].astype(jnp.int32)
        s_ali = jnp.minimum((s_base >> 3) << 3, _L - _WIN)
        cl = c_scr[ch * _CHUNK:(ch + 1) * _CHUNK, :]
        srange = (s_ali + lane).astype(jnp.float32)
        mask = (cl == srange).astype(jnp.bfloat16)
        hc = h[ch * _CHUNK:(ch + 1) * _CHUNK, :].astype(jnp.bfloat16)
        out_ref[0, pl.ds(s_ali, _WIN), :] += jax.lax.dot_general(
            mask, hc, (((0,), (0,)), ((), ())),
            preferred_element_type=jnp.float32)
        cnt_scr[pl.ds(s_ali, _WIN), :] += jax.lax.dot_general(
            mask, ones_c, (((0,), (0,)), ((), ())),
            preferred_element_type=jnp.float32)

    out_ref[0] = out_ref[0] / (cnt_scr[...] + 1e-9)


def _binom_loss(num_b, total_p, prior):
    logp = (gammaln(total_p + 1.0) - gammaln(num_b + 1.0)
            - gammaln(total_p - num_b + 1.0)
            + num_b * jnp.log(prior) + (total_p - num_b) * jnp.log1p(-prior))
    return -logp / total_p


def kernel(hidden, Wq, Wk, u):
    b, l, d = hidden.shape
    u3 = u.reshape(b, l, 1)
    pooled, nb = pl.pallas_call(
        _fused_kernel,
        grid=(b,),
        in_specs=[
            pl.BlockSpec((1, l, d), lambda i: (i, 0, 0)),
            pl.BlockSpec((d, d), lambda i: (0, 0)),
            pl.BlockSpec((d, d), lambda i: (0, 0)),
            pl.BlockSpec((1, l, 1), lambda i: (i, 0, 0)),
        ],
        out_specs=[
            pl.BlockSpec((1, l, d), lambda i: (i, 0, 0)),
            pl.BlockSpec((1, 1, 1), lambda i: (i, 0, 0)),
        ],
        out_shape=[
            jax.ShapeDtypeStruct((b, l, d), jnp.float32),
            jax.ShapeDtypeStruct((b, 1, 1), jnp.float32),
        ],
        scratch_shapes=[
            pltpu.VMEM((l, 1), jnp.float32),
            pltpu.VMEM((l, 1), jnp.float32),
        ],
        compiler_params=pltpu.CompilerParams(
            dimension_semantics=("parallel",)),
    )(hidden, Wq, Wk, u3)

    num_boundaries = jnp.sum(nb)
    total_positions = jnp.float32(b * l)
    loss = _binom_loss(num_boundaries, total_positions, _PRIOR)
    return pooled, loss, num_boundaries, total_positions
